# baseline (speedup 1.0000x reference)
import functools
import jax
import jax.numpy as jnp
from jax import lax
from jax.experimental import pallas as pl
from jax.experimental.pallas import tpu as pltpu
from jax.experimental.pallas import tpu_sc as plsc

E_DIM = 30
BATCH = 16384
NW = 32
B_PER_W = BATCH // NW
WAVES = B_PER_W // 16


def _rsqrt(x):
    i = plsc.bitcast(x, jnp.int32)
    i = 0x5F3759DF - lax.shift_right_arithmetic(i, 1)
    y = plsc.bitcast(i, jnp.float32)
    for _ in range(3):
        y = y * (1.5 - 0.5 * x * y * y)
    return y


@functools.partial(
    pl.kernel,
    mesh=plsc.VectorSubcoreMesh(core_axis_name="c", subcore_axis_name="s"),
    out_type=jax.ShapeDtypeStruct((BATCH,), jnp.float32),
    compiler_params=pltpu.CompilerParams(
        needs_layout_passes=False, use_tc_tiling_on_sc=True),
    scratch_types=[
        pltpu.VMEM((B_PER_W,), jnp.int32),
        pltpu.VMEM((B_PER_W,), jnp.int32),
        pltpu.VMEM((256, E_DIM), jnp.float32),
        pltpu.VMEM((256, E_DIM), jnp.float32),
        pltpu.VMEM((B_PER_W,), jnp.float32),
        pltpu.SemaphoreType.DMA,
    ],
)
def _sc_embed_dot(midx_hbm, lidx_hbm, wm_hbm, wl_hbm, out_hbm,
                  mi_v, li_v, m_w, l_w, r_v, sem):
    wid = lax.axis_index("s") * 2 + lax.axis_index("c")
    base = wid * B_PER_W

    pltpu.sync_copy(midx_hbm.at[pl.ds(base, B_PER_W)], mi_v)
    pltpu.sync_copy(lidx_hbm.at[pl.ds(base, B_PER_W)], li_v)

    iota16 = lax.broadcasted_iota(jnp.int32, (16,), 0)
    zeros = jnp.zeros((16,), jnp.float32)
    zeros_i = jnp.zeros((16,), jnp.int32)
    seven = jnp.full((16,), 7, jnp.int32)

    def wave_issue(c):
        b = c & 1
        o = c * 16
        ivm = mi_v[pl.ds(o, 16)]
        ivl = li_v[pl.ds(o, 16)]
        for l in range(16):
            lanemask = iota16 == l
            im = jnp.max(jnp.where(lanemask, ivm, 0))
            il = jnp.max(jnp.where(lanemask, ivl, 0))
            rm = pl.multiple_of(
                lax.shift_left(lax.shift_right_logical(im, 3), 3), 8)
            rl = pl.multiple_of(
                lax.shift_left(lax.shift_right_logical(il, 3), 3), 8)
            pltpu.async_copy(
                wm_hbm.at[pl.ds(rm, 8), :],
                m_w.at[pl.ds(128 * b + 8 * l, 8), :], sem)
            pltpu.async_copy(
                wl_hbm.at[pl.ds(rl, 8), :],
                l_w.at[pl.ds(128 * b + 8 * l, 8), :], sem)

    def wave_drain_compute(c):
        b = c & 1
        o = c * 16
        pltpu.make_async_copy(
            wm_hbm.at[pl.ds(0, 128), :],
            m_w.at[pl.ds(128 * b, 128), :], sem).wait()
        pltpu.make_async_copy(
            wl_hbm.at[pl.ds(0, 128), :],
            l_w.at[pl.ds(128 * b, 128), :], sem).wait()
        subm = mi_v[pl.ds(o, 16)] & seven
        subl = li_v[pl.ds(o, 16)] & seven
        ev = zeros_i + 128 * b + iota16 * 8
        md = zeros
        mm = zeros
        ll = zeros
        for j in range(E_DIM):
            jv = jnp.full((16,), j, jnp.int32)
            mv = plsc.load_gather(m_w, [ev + subm, jv])
            lv = plsc.load_gather(l_w, [ev + subl, jv])
            md = md + mv * lv
            mm = mm + mv * mv
            ll = ll + lv * lv
        r_v[pl.ds(o, 16)] = md * _rsqrt(mm * ll)

    wave_issue(0)

    def step(c, carry):
        @pl.when(c + 1 < WAVES)
        def _():
            wave_issue(c + 1)

        wave_drain_compute(c)
        return carry

    lax.fori_loop(0, WAVES, step, 0)

    pltpu.sync_copy(r_v, out_hbm.at[pl.ds(base, B_PER_W)])


def kernel(movie_batch, link_batch, W_movies, W_links):
    return _sc_embed_dot(
        movie_batch.astype(jnp.int32),
        link_batch.astype(jnp.int32),
        W_movies,
        W_links,
    )
